# Initial kernel scaffold; baseline (speedup 1.0000x reference)
import jax
import jax.numpy as jnp
from jax.experimental import pallas as pl


def kernel(sequence, sq_len, W_ih, W_hh, b_ih, b_hh, Wq, Wk, Wv, Wp, bp, Wh, bh):
    raise NotImplementedError("write your pallas kernel here")


# speedup vs baseline: 2.0454x; 2.0454x over previous
import jax
import jax.numpy as jnp
from jax.experimental import pallas as pl
from jax.experimental.pallas import tpu as pltpu

B, L, D, H = 128, 64, 256, 128
NUM_CLASSES, HASH_BITS = 100, 48


def _dranet_kernel(seq_ref, sl_col_ref, sl_row_ref,
                   Wih_t_ref, Whh_t_ref, b_ih_ref, b_hh_ref,
                   Wq_t_ref, Wk_ref, Wv_t_ref, Wp_t_ref, bp_ref,
                   Wh_t_ref, bh_ref,
                   pred_ref, hash_ref, att_ref,
                   gi_ref, v_ref):
    seq = seq_ref[...]
    x2 = seq.reshape(B * L, D)

    gi = jnp.dot(x2, Wih_t_ref[...], preferred_element_type=jnp.float32)
    gi_ref[...] = (gi + b_ih_ref[...]).reshape(B, L, 3 * H)

    v = jnp.dot(x2, Wv_t_ref[...], preferred_element_type=jnp.float32)
    v_ref[...] = jnp.maximum(v, 0.0).reshape(B, L, H)

    sl_col = sl_col_ref[...]
    b_hh = b_hh_ref[...]
    Whh_t = Whh_t_ref[...]

    def step(t, h):
        gi_t = gi_ref[:, pl.ds(t, 1), :].reshape(B, 3 * H)
        gh = jnp.dot(h, Whh_t, preferred_element_type=jnp.float32) + b_hh
        i_r = gi_t[:, :H]
        i_z = gi_t[:, H:2 * H]
        i_n = gi_t[:, 2 * H:]
        h_r = gh[:, :H]
        h_z = gh[:, H:2 * H]
        h_n = gh[:, 2 * H:]
        r = jax.nn.sigmoid(i_r + h_r)
        z = jax.nn.sigmoid(i_z + h_z)
        n = jnp.tanh(i_n + r * h_n)
        h_new = (1.0 - z) * n + z * h
        return jnp.where(t < sl_col, h_new, h)

    h0 = jnp.zeros((B, H), jnp.float32)
    hn = jax.lax.fori_loop(0, L, step, h0)

    query = jnp.dot(hn, Wq_t_ref[...], preferred_element_type=jnp.float32)
    qk = jnp.dot(query, Wk_ref[...], preferred_element_type=jnp.float32)
    dist = jnp.sum(seq * qk[:, None, :], axis=2)
    s = dist * (1.0 / jnp.sqrt(jnp.float32(H)))
    m = jnp.max(s, axis=1, keepdims=True)
    e = jnp.exp(s - m)
    pos_l = jax.lax.broadcasted_iota(jnp.int32, (B, L), 1)
    e = jnp.where(pos_l < sl_col, e, 0.0)
    att = e / jnp.sum(e, axis=1, keepdims=True)

    out = jnp.sum(att[:, :, None] * v_ref[...], axis=1) + query

    sl_row = sl_row_ref[...]
    iota_j = jax.lax.broadcasted_iota(jnp.int32, (B, B), 0)
    iota_i = jax.lax.broadcasted_iota(jnp.int32, (B, B), 1)
    before = (sl_col > sl_row) | ((sl_col == sl_row) & (iota_j < iota_i))
    rank_row = jnp.sum(before.astype(jnp.int32), axis=0, keepdims=True)
    perm = (iota_j == rank_row).astype(jnp.float32)
    att_ref[...] = jnp.dot(perm, att, preferred_element_type=jnp.float32)

    pred_ref[...] = jnp.dot(out, Wp_t_ref[...],
                            preferred_element_type=jnp.float32) + bp_ref[...]
    hash_ref[...] = jnp.tanh(jnp.dot(out, Wh_t_ref[...],
                                     preferred_element_type=jnp.float32)
                             + bh_ref[...])


@jax.jit
def kernel(sequence, sq_len, W_ih, W_hh, b_ih, b_hh, Wq, Wk, Wv, Wp, bp, Wh, bh):
    predict, hash_code, att_sq = pl.pallas_call(
        _dranet_kernel,
        out_shape=[
            jax.ShapeDtypeStruct((B, NUM_CLASSES), jnp.float32),
            jax.ShapeDtypeStruct((B, HASH_BITS), jnp.float32),
            jax.ShapeDtypeStruct((B, L), jnp.float32),
        ],
        scratch_shapes=[
            pltpu.VMEM((B, L, 3 * H), jnp.float32),
            pltpu.VMEM((B, L, H), jnp.float32),
        ],
        compiler_params=pltpu.CompilerParams(
            vmem_limit_bytes=100 * 1024 * 1024,
        ),
    )(sequence,
      sq_len.reshape(B, 1),
      sq_len.reshape(1, B),
      W_ih.T, W_hh.T,
      b_ih.reshape(1, -1), b_hh.reshape(1, -1),
      Wq.T, Wk, Wv.T,
      Wp.T, bp.reshape(1, -1),
      Wh.T, bh.reshape(1, -1))
    return predict, hash_code, att_sq


# speedup vs baseline: 2.5422x; 1.2429x over previous
import jax
import jax.numpy as jnp
from jax.experimental import pallas as pl
from jax.experimental.pallas import tpu as pltpu

B, L, D, H = 128, 64, 256, 128
NUM_CLASSES, HASH_BITS = 100, 48


def _dranet_kernel(seq_ref, sl_col_ref, sl_row_ref,
                   Wih_t_ref, Whh_t_ref, b_ih_ref, b_hh_ref,
                   Wq_t_ref, Wk_ref, Wv_t_ref, Wp_t_ref, bp_ref,
                   Wh_t_ref, bh_ref,
                   pred_ref, hash_ref, att_ref,
                   gi_ref, v_ref):
    seq = seq_ref[...]
    x2 = seq.reshape(L * B, D)

    b_ih = b_ih_ref[...]
    b_hh = b_hh_ref[...]
    b_comb = b_ih + jnp.where(
        jax.lax.broadcasted_iota(jnp.int32, (1, 3 * H), 1) < 2 * H, b_hh, 0.0)
    gi = jnp.dot(x2, Wih_t_ref[...], preferred_element_type=jnp.float32)
    gi_ref[...] = (gi + b_comb).reshape(L, B, 3 * H)

    v = jnp.dot(x2, Wv_t_ref[...], preferred_element_type=jnp.float32)
    v_ref[...] = jnp.maximum(v, 0.0).reshape(L, B, H)

    sl_col = sl_col_ref[...]
    b_hh_n = b_hh[:, 2 * H:]
    Whh_t = Whh_t_ref[...]

    def step(t, h):
        gi_t = gi_ref[pl.ds(t, 1)].reshape(B, 3 * H)
        gh = jnp.dot(h, Whh_t, preferred_element_type=jnp.float32)
        rz = jax.nn.sigmoid(gi_t[:, :2 * H] + gh[:, :2 * H])
        r = rz[:, :H]
        z = rz[:, H:]
        n = jnp.tanh(gi_t[:, 2 * H:] + r * (gh[:, 2 * H:] + b_hh_n))
        h_new = (1.0 - z) * n + z * h
        return jnp.where(t < sl_col, h_new, h)

    h0 = jnp.zeros((B, H), jnp.float32)
    hn = jax.lax.fori_loop(0, L, step, h0)

    query = jnp.dot(hn, Wq_t_ref[...], preferred_element_type=jnp.float32)
    qk = jnp.dot(query, Wk_ref[...], preferred_element_type=jnp.float32)
    dist = jnp.sum(seq * qk[None, :, :], axis=2)
    s = dist * (1.0 / jnp.sqrt(jnp.float32(H)))
    m = jnp.max(s, axis=0, keepdims=True)
    e = jnp.exp(s - m)
    pos_l = jax.lax.broadcasted_iota(jnp.int32, (L, B), 0)
    sl_row = sl_row_ref[...]
    e = jnp.where(pos_l < sl_row, e, 0.0)
    att = e / jnp.sum(e, axis=0, keepdims=True)

    out = jnp.sum(att[:, :, None] * v_ref[...], axis=0) + query

    iota_j = jax.lax.broadcasted_iota(jnp.int32, (B, B), 0)
    iota_i = jax.lax.broadcasted_iota(jnp.int32, (B, B), 1)
    before = (sl_col > sl_row) | ((sl_col == sl_row) & (iota_j < iota_i))
    rank_row = jnp.sum(before.astype(jnp.int32), axis=0, keepdims=True)
    perm = (iota_j == rank_row).astype(jnp.float32)
    att_ref[...] = jnp.dot(perm, att.T, preferred_element_type=jnp.float32)

    pred_ref[...] = jnp.dot(out, Wp_t_ref[...],
                            preferred_element_type=jnp.float32) + bp_ref[...]
    hash_ref[...] = jnp.tanh(jnp.dot(out, Wh_t_ref[...],
                                     preferred_element_type=jnp.float32)
                             + bh_ref[...])


@jax.jit
def kernel(sequence, sq_len, W_ih, W_hh, b_ih, b_hh, Wq, Wk, Wv, Wp, bp, Wh, bh):
    predict, hash_code, att_sq = pl.pallas_call(
        _dranet_kernel,
        out_shape=[
            jax.ShapeDtypeStruct((B, NUM_CLASSES), jnp.float32),
            jax.ShapeDtypeStruct((B, HASH_BITS), jnp.float32),
            jax.ShapeDtypeStruct((B, L), jnp.float32),
        ],
        scratch_shapes=[
            pltpu.VMEM((L, B, 3 * H), jnp.float32),
            pltpu.VMEM((L, B, H), jnp.float32),
        ],
        compiler_params=pltpu.CompilerParams(
            vmem_limit_bytes=100 * 1024 * 1024,
        ),
    )(jnp.swapaxes(sequence, 0, 1),
      sq_len.reshape(B, 1),
      sq_len.reshape(1, B),
      W_ih.T, W_hh.T,
      b_ih.reshape(1, -1), b_hh.reshape(1, -1),
      Wq.T, Wk, Wv.T,
      Wp.T, bp.reshape(1, -1),
      Wh.T, bh.reshape(1, -1))
    return predict, hash_code, att_sq


# speedup vs baseline: 2.5807x; 1.0151x over previous
import jax
import jax.numpy as jnp
from jax.experimental import pallas as pl
from jax.experimental.pallas import tpu as pltpu

B, L, D, H = 128, 64, 256, 128
NUM_CLASSES, HASH_BITS = 100, 48


def _dranet_kernel(seq_ref, sl_col_ref, sl_row_ref,
                   Wih_t_ref, Whh_t_ref, b_ih_ref, b_hh_ref,
                   Wq_t_ref, Wk_ref, Wv_t_ref, Wp_t_ref, bp_ref,
                   Wh_t_ref, bh_ref,
                   pred_ref, hash_ref, att_ref,
                   gi_ref, v_ref):
    seq = seq_ref[...]
    x2 = seq.reshape(L * B, D)

    b_ih = b_ih_ref[...]
    b_hh = b_hh_ref[...]
    b_comb = b_ih + jnp.where(
        jax.lax.broadcasted_iota(jnp.int32, (1, 3 * H), 1) < 2 * H, b_hh, 0.0)
    gi = jnp.dot(x2, Wih_t_ref[...], preferred_element_type=jnp.float32)
    gi_ref[...] = (gi + b_comb).reshape(L, B, 3 * H)

    v = jnp.dot(x2, Wv_t_ref[...], preferred_element_type=jnp.float32)
    v_ref[...] = jnp.maximum(v, 0.0).reshape(L, B, H)

    sl_col = sl_col_ref[...]
    b_hh_n = b_hh[:, 2 * H:]
    Whh_t = Whh_t_ref[...]

    B2 = B // 2
    sl_a = sl_col[:B2]
    sl_b = sl_col[B2:]

    def half_step(gi_h, gh, h, valid):
        rz = jax.nn.sigmoid(gi_h[:, :2 * H] + gh[:, :2 * H])
        r = rz[:, :H]
        z = rz[:, H:]
        n = jnp.tanh(gi_h[:, 2 * H:] + r * (gh[:, 2 * H:] + b_hh_n))
        h_new = (1.0 - z) * n + z * h
        return jnp.where(valid, h_new, h)

    def step(t, carry):
        ha, hb = carry
        gi_t = gi_ref[pl.ds(t, 1)].reshape(B, 3 * H)
        gha = jnp.dot(ha, Whh_t, preferred_element_type=jnp.float32)
        ghb = jnp.dot(hb, Whh_t, preferred_element_type=jnp.float32)
        ha = half_step(gi_t[:B2], gha, ha, t < sl_a)
        hb = half_step(gi_t[B2:], ghb, hb, t < sl_b)
        return ha, hb

    h0 = jnp.zeros((B2, H), jnp.float32)
    hna, hnb = jax.lax.fori_loop(0, L, step, (h0, h0))
    hn = jnp.concatenate([hna, hnb], axis=0)

    query = jnp.dot(hn, Wq_t_ref[...], preferred_element_type=jnp.float32)
    qk = jnp.dot(query, Wk_ref[...], preferred_element_type=jnp.float32)
    dist = jnp.sum(seq * qk[None, :, :], axis=2)
    s = dist * (1.0 / jnp.sqrt(jnp.float32(H)))
    m = jnp.max(s, axis=0, keepdims=True)
    e = jnp.exp(s - m)
    pos_l = jax.lax.broadcasted_iota(jnp.int32, (L, B), 0)
    sl_row = sl_row_ref[...]
    e = jnp.where(pos_l < sl_row, e, 0.0)
    att = e / jnp.sum(e, axis=0, keepdims=True)

    out = jnp.sum(att[:, :, None] * v_ref[...], axis=0) + query

    iota_j = jax.lax.broadcasted_iota(jnp.int32, (B, B), 0)
    iota_i = jax.lax.broadcasted_iota(jnp.int32, (B, B), 1)
    before = (sl_col > sl_row) | ((sl_col == sl_row) & (iota_j < iota_i))
    rank_row = jnp.sum(before.astype(jnp.int32), axis=0, keepdims=True)
    perm = (iota_j == rank_row).astype(jnp.float32)
    att_ref[...] = jnp.dot(perm, att.T, preferred_element_type=jnp.float32)

    pred_ref[...] = jnp.dot(out, Wp_t_ref[...],
                            preferred_element_type=jnp.float32) + bp_ref[...]
    hash_ref[...] = jnp.tanh(jnp.dot(out, Wh_t_ref[...],
                                     preferred_element_type=jnp.float32)
                             + bh_ref[...])


@jax.jit
def kernel(sequence, sq_len, W_ih, W_hh, b_ih, b_hh, Wq, Wk, Wv, Wp, bp, Wh, bh):
    predict, hash_code, att_sq = pl.pallas_call(
        _dranet_kernel,
        out_shape=[
            jax.ShapeDtypeStruct((B, NUM_CLASSES), jnp.float32),
            jax.ShapeDtypeStruct((B, HASH_BITS), jnp.float32),
            jax.ShapeDtypeStruct((B, L), jnp.float32),
        ],
        scratch_shapes=[
            pltpu.VMEM((L, B, 3 * H), jnp.float32),
            pltpu.VMEM((L, B, H), jnp.float32),
        ],
        compiler_params=pltpu.CompilerParams(
            vmem_limit_bytes=100 * 1024 * 1024,
        ),
    )(jnp.swapaxes(sequence, 0, 1),
      sq_len.reshape(B, 1),
      sq_len.reshape(1, B),
      W_ih.T, W_hh.T,
      b_ih.reshape(1, -1), b_hh.reshape(1, -1),
      Wq.T, Wk, Wv.T,
      Wp.T, bp.reshape(1, -1),
      Wh.T, bh.reshape(1, -1))
    return predict, hash_code, att_sq


# speedup vs baseline: 2.6333x; 1.0204x over previous
import jax
import jax.numpy as jnp
from jax.experimental import pallas as pl
from jax.experimental.pallas import tpu as pltpu

B, L, D, H = 128, 64, 256, 128
NUM_CLASSES, HASH_BITS = 100, 48


def _dranet_kernel(seq_ref, sl_col_ref, sl_row_ref,
                   Wih_t_ref, Whh_t_ref, b_ih_ref, b_hh_ref,
                   Wq_t_ref, Wk_ref, Wv_t_ref, Wp_t_ref, bp_ref,
                   Wh_t_ref, bh_ref,
                   pred_ref, hash_ref, att_ref,
                   gi_ref, v_ref):
    seq = seq_ref[...]
    x2 = seq.reshape(L * B, D)

    b_ih = b_ih_ref[...]
    b_hh = b_hh_ref[...]
    b_comb = b_ih + jnp.where(
        jax.lax.broadcasted_iota(jnp.int32, (1, 3 * H), 1) < 2 * H, b_hh, 0.0)
    gi = jnp.dot(x2, Wih_t_ref[...], preferred_element_type=jnp.float32)
    gi_ref[...] = (gi + b_comb).reshape(L, B, 3 * H)

    v = jnp.dot(x2, Wv_t_ref[...], preferred_element_type=jnp.float32)
    v_ref[...] = jnp.maximum(v, 0.0).reshape(L, B, H)

    sl_col = sl_col_ref[...]
    b_hh_n = b_hh[:, 2 * H:]
    Whh_t = Whh_t_ref[...]

    B2 = B // 2
    sl_a = sl_col[:B2]
    sl_b = sl_col[B2:]

    def half_step(gi_h, gh, h, valid):
        rz = jax.nn.sigmoid(gi_h[:, :2 * H] + gh[:, :2 * H])
        r = rz[:, :H]
        z = rz[:, H:]
        n = jnp.tanh(gi_h[:, 2 * H:] + r * (gh[:, 2 * H:] + b_hh_n))
        h_new = (1.0 - z) * n + z * h
        return jnp.where(valid, h_new, h)

    def step(i, carry):
        ha, hb = carry
        for u in range(2):
            t = i * 2 + u
            gi_t = gi_ref[pl.ds(t, 1)].reshape(B, 3 * H)
            gha = jnp.dot(ha, Whh_t, preferred_element_type=jnp.float32)
            ghb = jnp.dot(hb, Whh_t, preferred_element_type=jnp.float32)
            ha = half_step(gi_t[:B2], gha, ha, t < sl_a)
            hb = half_step(gi_t[B2:], ghb, hb, t < sl_b)
        return ha, hb

    h0 = jnp.zeros((B2, H), jnp.float32)
    hna, hnb = jax.lax.fori_loop(0, L // 2, step, (h0, h0))
    hn = jnp.concatenate([hna, hnb], axis=0)

    query = jnp.dot(hn, Wq_t_ref[...], preferred_element_type=jnp.float32)
    qk = jnp.dot(query, Wk_ref[...], preferred_element_type=jnp.float32)
    dist = jnp.sum(seq * qk[None, :, :], axis=2)
    s = dist * (1.0 / jnp.sqrt(jnp.float32(H)))
    m = jnp.max(s, axis=0, keepdims=True)
    e = jnp.exp(s - m)
    pos_l = jax.lax.broadcasted_iota(jnp.int32, (L, B), 0)
    sl_row = sl_row_ref[...]
    e = jnp.where(pos_l < sl_row, e, 0.0)
    att = e / jnp.sum(e, axis=0, keepdims=True)

    out = jnp.sum(att[:, :, None] * v_ref[...], axis=0) + query

    iota_j = jax.lax.broadcasted_iota(jnp.int32, (B, B), 0)
    iota_i = jax.lax.broadcasted_iota(jnp.int32, (B, B), 1)
    before = (sl_col > sl_row) | ((sl_col == sl_row) & (iota_j < iota_i))
    rank_row = jnp.sum(before.astype(jnp.int32), axis=0, keepdims=True)
    perm = (iota_j == rank_row).astype(jnp.float32)
    att_ref[...] = jnp.dot(perm, att.T, preferred_element_type=jnp.float32)

    pred_ref[...] = jnp.dot(out, Wp_t_ref[...],
                            preferred_element_type=jnp.float32) + bp_ref[...]
    hash_ref[...] = jnp.tanh(jnp.dot(out, Wh_t_ref[...],
                                     preferred_element_type=jnp.float32)
                             + bh_ref[...])


@jax.jit
def kernel(sequence, sq_len, W_ih, W_hh, b_ih, b_hh, Wq, Wk, Wv, Wp, bp, Wh, bh):
    predict, hash_code, att_sq = pl.pallas_call(
        _dranet_kernel,
        out_shape=[
            jax.ShapeDtypeStruct((B, NUM_CLASSES), jnp.float32),
            jax.ShapeDtypeStruct((B, HASH_BITS), jnp.float32),
            jax.ShapeDtypeStruct((B, L), jnp.float32),
        ],
        scratch_shapes=[
            pltpu.VMEM((L, B, 3 * H), jnp.float32),
            pltpu.VMEM((L, B, H), jnp.float32),
        ],
        compiler_params=pltpu.CompilerParams(
            vmem_limit_bytes=100 * 1024 * 1024,
        ),
    )(jnp.swapaxes(sequence, 0, 1),
      sq_len.reshape(B, 1),
      sq_len.reshape(1, B),
      W_ih.T, W_hh.T,
      b_ih.reshape(1, -1), b_hh.reshape(1, -1),
      Wq.T, Wk, Wv.T,
      Wp.T, bp.reshape(1, -1),
      Wh.T, bh.reshape(1, -1))
    return predict, hash_code, att_sq


# speedup vs baseline: 2.6578x; 1.0093x over previous
import jax
import jax.numpy as jnp
from jax.experimental import pallas as pl
from jax.experimental.pallas import tpu as pltpu

B, L, D, H = 128, 64, 256, 128
NUM_CLASSES, HASH_BITS = 100, 48


def _dranet_kernel(seq_ref, sl_col_ref, sl_row_ref,
                   Wih_t_ref, Whh_t_ref, b_ih_ref, b_hh_ref,
                   Wq_t_ref, Wk_ref, Wv_t_ref, Wp_t_ref, bp_ref,
                   Wh_t_ref, bh_ref,
                   pred_ref, hash_ref, att_ref,
                   gi_ref, v_ref):
    seq = seq_ref[...]
    x2 = seq.reshape(L * B, D)

    b_ih = b_ih_ref[...]
    b_hh = b_hh_ref[...]
    b_comb = b_ih + jnp.where(
        jax.lax.broadcasted_iota(jnp.int32, (1, 3 * H), 1) < 2 * H, b_hh, 0.0)
    gi = jnp.dot(x2, Wih_t_ref[...], preferred_element_type=jnp.float32)
    gi_ref[...] = (gi + b_comb).reshape(L, B, 3 * H)

    v = jnp.dot(x2, Wv_t_ref[...], preferred_element_type=jnp.float32)
    v_ref[...] = jnp.maximum(v, 0.0).reshape(L, B, H)

    sl_col = sl_col_ref[...]
    b_hh_n = b_hh[:, 2 * H:]
    Whh_t = Whh_t_ref[...].astype(jnp.bfloat16)

    B2 = B // 2
    sl_a = sl_col[:B2]
    sl_b = sl_col[B2:]

    def half_step(gi_h, gh, h, valid):
        rz = jax.nn.sigmoid(gi_h[:, :2 * H] + gh[:, :2 * H])
        r = rz[:, :H]
        z = rz[:, H:]
        n = jnp.tanh(gi_h[:, 2 * H:] + r * (gh[:, 2 * H:] + b_hh_n))
        h_new = (1.0 - z) * n + z * h
        return jnp.where(valid, h_new, h)

    def step(i, carry):
        ha, hb = carry
        for u in range(2):
            t = i * 2 + u
            gi_t = gi_ref[pl.ds(t, 1)].reshape(B, 3 * H)
            gha = jnp.dot(ha.astype(jnp.bfloat16), Whh_t,
                          preferred_element_type=jnp.float32)
            ghb = jnp.dot(hb.astype(jnp.bfloat16), Whh_t,
                          preferred_element_type=jnp.float32)
            ha = half_step(gi_t[:B2], gha, ha, t < sl_a)
            hb = half_step(gi_t[B2:], ghb, hb, t < sl_b)
        return ha, hb

    h0 = jnp.zeros((B2, H), jnp.float32)
    hna, hnb = jax.lax.fori_loop(0, L // 2, step, (h0, h0))
    hn = jnp.concatenate([hna, hnb], axis=0)

    query = jnp.dot(hn, Wq_t_ref[...], preferred_element_type=jnp.float32)
    qk = jnp.dot(query, Wk_ref[...], preferred_element_type=jnp.float32)
    dist = jnp.sum(seq * qk[None, :, :], axis=2)
    s = dist * (1.0 / jnp.sqrt(jnp.float32(H)))
    m = jnp.max(s, axis=0, keepdims=True)
    e = jnp.exp(s - m)
    pos_l = jax.lax.broadcasted_iota(jnp.int32, (L, B), 0)
    sl_row = sl_row_ref[...]
    e = jnp.where(pos_l < sl_row, e, 0.0)
    att = e / jnp.sum(e, axis=0, keepdims=True)

    out = jnp.sum(att[:, :, None] * v_ref[...], axis=0) + query

    iota_j = jax.lax.broadcasted_iota(jnp.int32, (B, B), 0)
    iota_i = jax.lax.broadcasted_iota(jnp.int32, (B, B), 1)
    before = (sl_col > sl_row) | ((sl_col == sl_row) & (iota_j < iota_i))
    rank_row = jnp.sum(before.astype(jnp.int32), axis=0, keepdims=True)
    perm = (iota_j == rank_row).astype(jnp.float32)
    att_ref[...] = jnp.dot(perm, att.T, preferred_element_type=jnp.float32)

    pred_ref[...] = jnp.dot(out, Wp_t_ref[...],
                            preferred_element_type=jnp.float32) + bp_ref[...]
    hash_ref[...] = jnp.tanh(jnp.dot(out, Wh_t_ref[...],
                                     preferred_element_type=jnp.float32)
                             + bh_ref[...])


@jax.jit
def kernel(sequence, sq_len, W_ih, W_hh, b_ih, b_hh, Wq, Wk, Wv, Wp, bp, Wh, bh):
    predict, hash_code, att_sq = pl.pallas_call(
        _dranet_kernel,
        out_shape=[
            jax.ShapeDtypeStruct((B, NUM_CLASSES), jnp.float32),
            jax.ShapeDtypeStruct((B, HASH_BITS), jnp.float32),
            jax.ShapeDtypeStruct((B, L), jnp.float32),
        ],
        scratch_shapes=[
            pltpu.VMEM((L, B, 3 * H), jnp.float32),
            pltpu.VMEM((L, B, H), jnp.float32),
        ],
        compiler_params=pltpu.CompilerParams(
            vmem_limit_bytes=100 * 1024 * 1024,
        ),
    )(jnp.swapaxes(sequence, 0, 1),
      sq_len.reshape(B, 1),
      sq_len.reshape(1, B),
      W_ih.T, W_hh.T,
      b_ih.reshape(1, -1), b_hh.reshape(1, -1),
      Wq.T, Wk, Wv.T,
      Wp.T, bp.reshape(1, -1),
      Wh.T, bh.reshape(1, -1))
    return predict, hash_code, att_sq
